# initial kernel scaffold (unmeasured)
import functools

import jax
import jax.numpy as jnp
from jax import lax
from jax.experimental import pallas as pl
from jax.experimental.pallas import tpu as pltpu

N_DEV = 32
B, S, H, Dh, Dr = 4, 256, 32, 128, 64
D = 4096


def _rs_body(kv_ref, out_ref, recv_ref, send_sems, recv_sems, copy_sem):
    my = lax.axis_index("i")

    cp = pltpu.make_async_copy(kv_ref.at[my], recv_ref.at[my], copy_sem)
    cp.start()

    sends = []
    for p in range(N_DEV):
        rdma = pltpu.make_async_remote_copy(
            src_ref=kv_ref.at[p],
            dst_ref=recv_ref.at[my],
            send_sem=send_sems.at[p],
            recv_sem=recv_sems.at[my],
            device_id=(p,),
            device_id_type=pl.DeviceIdType.MESH,
        )
        sends.append(rdma)

        @pl.when(my != p)
        def _():
            rdma.start()

    cp.wait()

    for p in range(N_DEV):
        recv = pltpu.make_async_remote_copy(
            src_ref=kv_ref.at[p],
            dst_ref=recv_ref.at[p],
            send_sem=send_sems.at[p],
            recv_sem=recv_sems.at[p],
            device_id=(p,),
            device_id_type=pl.DeviceIdType.MESH,
        )

        @pl.when(my != p)
        def _():
            recv.wait_recv()

    acc = recv_ref[0].astype(jnp.float32)
    for p in range(1, N_DEV):
        acc = acc + recv_ref[p].astype(jnp.float32)
    out_ref[...] = acc.astype(jnp.bfloat16)

    for p in range(N_DEV):

        @pl.when(my != p)
        def _(rdma=sends[p]):
            rdma.wait_send()


def _reduce_scatter_kv(kv):
    return pl.pallas_call(
        _rs_body,
        out_shape=jax.ShapeDtypeStruct((2, B * S, Dh), jnp.bfloat16),
        in_specs=[pl.BlockSpec(memory_space=pltpu.VMEM)],
        out_specs=pl.BlockSpec(memory_space=pltpu.VMEM),
        scratch_shapes=[
            pltpu.VMEM((N_DEV, 2, B * S, Dh), jnp.bfloat16),
            pltpu.SemaphoreType.DMA((N_DEV,)),
            pltpu.SemaphoreType.DMA((N_DEV,)),
            pltpu.SemaphoreType.DMA,
        ],
        compiler_params=pltpu.CompilerParams(collective_id=0),
    )(kv)


def _ag_body(o_ref, out_ref, send_sems, recv_sems, copy_sem):
    my = lax.axis_index("i")

    cp = pltpu.make_async_copy(o_ref, out_ref.at[my], copy_sem)
    cp.start()

    sends = []
    for p in range(N_DEV):
        rdma = pltpu.make_async_remote_copy(
            src_ref=o_ref,
            dst_ref=out_ref.at[my],
            send_sem=send_sems.at[p],
            recv_sem=recv_sems.at[my],
            device_id=(p,),
            device_id_type=pl.DeviceIdType.MESH,
        )
        sends.append(rdma)

        @pl.when(my != p)
        def _():
            rdma.start()

    cp.wait()

    for p in range(N_DEV):
        recv = pltpu.make_async_remote_copy(
            src_ref=o_ref,
            dst_ref=out_ref.at[p],
            send_sem=send_sems.at[p],
            recv_sem=recv_sems.at[p],
            device_id=(p,),
            device_id_type=pl.DeviceIdType.MESH,
        )

        @pl.when(my != p)
        def _():
            recv.wait_recv()

    for p in range(N_DEV):

        @pl.when(my != p)
        def _(rdma=sends[p]):
            rdma.wait_send()


def _all_gather_o(o):
    return pl.pallas_call(
        _ag_body,
        out_shape=jax.ShapeDtypeStruct((N_DEV, B * S, Dh), jnp.bfloat16),
        in_specs=[pl.BlockSpec(memory_space=pltpu.VMEM)],
        out_specs=pl.BlockSpec(memory_space=pltpu.VMEM),
        scratch_shapes=[
            pltpu.SemaphoreType.DMA((N_DEV,)),
            pltpu.SemaphoreType.DMA((N_DEV,)),
            pltpu.SemaphoreType.DMA,
        ],
        compiler_params=pltpu.CompilerParams(collective_id=1),
    )(o)


def kernel(x, Wdkv, Wuk, Wuv, Wq, Wqr, Wkr, Wo):
    my = lax.axis_index("i")
    bf16 = jnp.bfloat16
    f32 = jnp.float32

    xb = x.astype(bf16).reshape(B * S, D)
    c = xb @ Wdkv.astype(bf16)
    Kp = c @ Wuk.astype(bf16)
    Vp = c @ Wuv.astype(bf16)

    kv = jnp.stack(
        [
            Kp.reshape(B * S, H, Dh).transpose(1, 0, 2),
            Vp.reshape(B * S, H, Dh).transpose(1, 0, 2),
        ],
        axis=1,
    )
    kvh = _reduce_scatter_kv(kv)
    K_h = kvh[0].reshape(B, S, Dh)
    V_h = kvh[1].reshape(B, S, Dh)

    Wq_h = lax.dynamic_slice_in_dim(Wq, my * Dh, Dh, axis=1).astype(bf16)
    Q_h = (xb @ Wq_h).reshape(B, S, Dh)
    Wqr_h = lax.dynamic_slice_in_dim(Wqr, my * Dr, Dr, axis=1).astype(bf16)
    Qr_h = (xb @ Wqr_h).reshape(B, S, Dr)
    Kr = (xb @ Wkr.astype(bf16)).reshape(B, S, Dr)

    scale = (Dh + Dr) ** -0.5
    scores = (
        jnp.einsum("bsd,btd->bst", Q_h, K_h, preferred_element_type=f32)
        + jnp.einsum("bsd,btd->bst", Qr_h, Kr, preferred_element_type=f32)
    ) * scale
    P = jax.nn.softmax(scores, axis=-1).astype(bf16)
    O_h = jnp.einsum("bst,btd->bsd", P, V_h, preferred_element_type=f32)
    O_h = O_h.astype(bf16).reshape(B * S, Dh)

    O_all = _all_gather_o(O_h)
    O_full = O_all.transpose(1, 0, 2).reshape(B * S, H * Dh)

    out = jnp.dot(O_full, Wo.astype(bf16), preferred_element_type=f32)
    return out.reshape(B, S, D).astype(f32)


# baseline (device time: 572336 ns/iter reference)
import functools

import jax
import jax.numpy as jnp
from jax import lax
from jax.experimental import pallas as pl
from jax.experimental.pallas import tpu as pltpu

N_DEV = 32
B, S, H, Dh, Dr = 4, 256, 32, 128, 64
D = 4096


def _rs_body(kv_ref, out_ref, recv_ref, send_sems, recv_sems, copy_sem):
    my = lax.axis_index("i")

    cp = pltpu.make_async_copy(kv_ref.at[my], recv_ref.at[my], copy_sem)
    cp.start()

    sends = []
    for p in range(N_DEV):
        rdma = pltpu.make_async_remote_copy(
            src_ref=kv_ref.at[p],
            dst_ref=recv_ref.at[my],
            send_sem=send_sems.at[p],
            recv_sem=recv_sems.at[my],
            device_id=(p,),
            device_id_type=pl.DeviceIdType.MESH,
        )
        sends.append(rdma)

        @pl.when(my != p)
        def _():
            rdma.start()

    cp.wait()

    for p in range(N_DEV):
        recv = pltpu.make_async_remote_copy(
            src_ref=kv_ref.at[p],
            dst_ref=recv_ref.at[p],
            send_sem=send_sems.at[p],
            recv_sem=recv_sems.at[p],
            device_id=(p,),
            device_id_type=pl.DeviceIdType.MESH,
        )

        @pl.when(my != p)
        def _():
            recv.wait_recv()

    acc = recv_ref[0].astype(jnp.float32)
    for p in range(1, N_DEV):
        acc = acc + recv_ref[p].astype(jnp.float32)
    out_ref[...] = acc.astype(jnp.bfloat16)

    for p in range(N_DEV):

        @pl.when(my != p)
        def _(rdma=sends[p]):
            rdma.wait_send()


def _reduce_scatter_kv(kv):
    return pl.pallas_call(
        _rs_body,
        out_shape=jax.ShapeDtypeStruct((2, B * S, Dh), jnp.bfloat16),
        in_specs=[pl.BlockSpec(memory_space=pltpu.VMEM)],
        out_specs=pl.BlockSpec(memory_space=pltpu.VMEM),
        scratch_shapes=[
            pltpu.VMEM((N_DEV, 2, B * S, Dh), jnp.bfloat16),
            pltpu.SemaphoreType.DMA((N_DEV,)),
            pltpu.SemaphoreType.DMA((N_DEV,)),
            pltpu.SemaphoreType.DMA,
        ],
    )(kv)


def _ag_body(o_ref, out_ref, send_sems, recv_sems, copy_sem):
    my = lax.axis_index("i")

    cp = pltpu.make_async_copy(o_ref, out_ref.at[my], copy_sem)
    cp.start()

    sends = []
    for p in range(N_DEV):
        rdma = pltpu.make_async_remote_copy(
            src_ref=o_ref,
            dst_ref=out_ref.at[my],
            send_sem=send_sems.at[p],
            recv_sem=recv_sems.at[my],
            device_id=(p,),
            device_id_type=pl.DeviceIdType.MESH,
        )
        sends.append(rdma)

        @pl.when(my != p)
        def _():
            rdma.start()

    cp.wait()

    for p in range(N_DEV):
        recv = pltpu.make_async_remote_copy(
            src_ref=o_ref,
            dst_ref=out_ref.at[p],
            send_sem=send_sems.at[p],
            recv_sem=recv_sems.at[p],
            device_id=(p,),
            device_id_type=pl.DeviceIdType.MESH,
        )

        @pl.when(my != p)
        def _():
            recv.wait_recv()

    for p in range(N_DEV):

        @pl.when(my != p)
        def _(rdma=sends[p]):
            rdma.wait_send()


def _all_gather_o(o):
    return pl.pallas_call(
        _ag_body,
        out_shape=jax.ShapeDtypeStruct((N_DEV, B * S, Dh), jnp.bfloat16),
        in_specs=[pl.BlockSpec(memory_space=pltpu.VMEM)],
        out_specs=pl.BlockSpec(memory_space=pltpu.VMEM),
        scratch_shapes=[
            pltpu.SemaphoreType.DMA((N_DEV,)),
            pltpu.SemaphoreType.DMA((N_DEV,)),
            pltpu.SemaphoreType.DMA,
        ],
    )(o)


def kernel(x, Wdkv, Wuk, Wuv, Wq, Wqr, Wkr, Wo):
    my = lax.axis_index("i")
    bf16 = jnp.bfloat16
    f32 = jnp.float32

    xb = x.astype(bf16).reshape(B * S, D)
    c = xb @ Wdkv.astype(bf16)
    Kp = c @ Wuk.astype(bf16)
    Vp = c @ Wuv.astype(bf16)

    kv = jnp.stack(
        [
            Kp.reshape(B * S, H, Dh).transpose(1, 0, 2),
            Vp.reshape(B * S, H, Dh).transpose(1, 0, 2),
        ],
        axis=1,
    )
    kvh = _reduce_scatter_kv(kv)
    K_h = kvh[0].reshape(B, S, Dh)
    V_h = kvh[1].reshape(B, S, Dh)

    Wq_h = lax.dynamic_slice_in_dim(Wq, my * Dh, Dh, axis=1).astype(bf16)
    Q_h = (xb @ Wq_h).reshape(B, S, Dh)
    Wqr_h = lax.dynamic_slice_in_dim(Wqr, my * Dr, Dr, axis=1).astype(bf16)
    Qr_h = (xb @ Wqr_h).reshape(B, S, Dr)
    Kr = (xb @ Wkr.astype(bf16)).reshape(B, S, Dr)

    scale = (Dh + Dr) ** -0.5
    scores = (
        jnp.einsum("bsd,btd->bst", Q_h, K_h, preferred_element_type=f32)
        + jnp.einsum("bsd,btd->bst", Qr_h, Kr, preferred_element_type=f32)
    ) * scale
    P = jax.nn.softmax(scores, axis=-1).astype(bf16)
    O_h = jnp.einsum("bst,btd->bsd", P, V_h, preferred_element_type=f32)
    O_h = O_h.astype(bf16).reshape(B * S, Dh)

    O_all = _all_gather_o(O_h)
    O_full = O_all.transpose(1, 0, 2).reshape(B * S, H * Dh)

    out = jnp.dot(O_full, Wo.astype(bf16), preferred_element_type=f32)
    return out.reshape(B, S, D).astype(f32)
